# baseline (speedup 1.0000x reference)
import jax
import jax.numpy as jnp
from jax.experimental import pallas as pl
from jax.experimental.pallas import tpu as pltpu

_BP = (-2.0, -1.0, 0.0, 1.0, 2.0)
_COEFFS = (
    (0.5, -1.0, 0.25, 0.10),
    (0.0, 1.0, -0.50, 0.20),
    (0.3, 0.7, 0.10, -0.15),
    (-0.2, 0.4, 0.60, 0.05),
    (1.0, -0.3, 0.20, 0.01),
)

_BM = 1024
_BN = 2048


def _piecewise_val(v):
    m0 = v <= _BP[1]
    m1 = v <= _BP[2]
    m2 = v <= _BP[3]
    m3 = v <= _BP[4]

    def sel(k):
        c = _COEFFS
        return jnp.where(
            m0, c[0][k],
            jnp.where(m1, c[1][k], jnp.where(m2, c[2][k], jnp.where(m3, c[3][k], c[4][k]))),
        )

    c0, c1, c2, c3 = sel(0), sel(1), sel(2), sel(3)
    val = ((c3 * v + c2) * v + c1) * v + c0
    return jnp.where(v < _BP[0], 0.0, val)


def _tile_kernel(x_ref, o_ref):
    o_ref[...] = _piecewise_val(x_ref[...]).T


def kernel(x, w, b):
    del w, b
    B, F = x.shape
    grid = (B // _BM, F // _BN)
    return pl.pallas_call(
        _tile_kernel,
        grid=grid,
        in_specs=[pl.BlockSpec((_BM, _BN), lambda i, j: (i, j))],
        out_specs=pl.BlockSpec((_BN, _BM), lambda i, j: (j, i)),
        out_shape=jax.ShapeDtypeStruct((F, B), x.dtype),
        compiler_params=pltpu.CompilerParams(
            dimension_semantics=("parallel", "arbitrary"),
        ),
    )(x)


# speedup vs baseline: 1.0086x; 1.0086x over previous
import jax
import jax.numpy as jnp
from jax.experimental import pallas as pl
from jax.experimental.pallas import tpu as pltpu

_BP = (-2.0, -1.0, 0.0, 1.0, 2.0)
_COEFFS = (
    (0.5, -1.0, 0.25, 0.10),
    (0.0, 1.0, -0.50, 0.20),
    (0.3, 0.7, 0.10, -0.15),
    (-0.2, 0.4, 0.60, 0.05),
    (1.0, -0.3, 0.20, 0.01),
)

_BM = 1024
_BN = 1024


def _piecewise_val(v):
    m0 = v <= _BP[1]
    m1 = v <= _BP[2]
    m2 = v <= _BP[3]
    m3 = v <= _BP[4]

    def sel(k):
        c = _COEFFS
        return jnp.where(
            m0, c[0][k],
            jnp.where(m1, c[1][k], jnp.where(m2, c[2][k], jnp.where(m3, c[3][k], c[4][k]))),
        )

    c0, c1, c2, c3 = sel(0), sel(1), sel(2), sel(3)
    val = ((c3 * v + c2) * v + c1) * v + c0
    return jnp.where(v < _BP[0], 0.0, val)


def _tile_kernel(x_ref, o_ref):
    o_ref[...] = _piecewise_val(x_ref[...]).T


def kernel(x, w, b):
    del w, b
    B, F = x.shape
    grid = (B // _BM, F // _BN)
    return pl.pallas_call(
        _tile_kernel,
        grid=grid,
        in_specs=[pl.BlockSpec((_BM, _BN), lambda i, j: (i, j))],
        out_specs=pl.BlockSpec((_BN, _BM), lambda i, j: (j, i)),
        out_shape=jax.ShapeDtypeStruct((F, B), x.dtype),
        compiler_params=pltpu.CompilerParams(
            dimension_semantics=("parallel", "parallel"),
        ),
    )(x)
